# baseline (speedup 1.0000x reference)
import functools

import jax
import jax.numpy as jnp
from jax import lax
from jax.experimental import pallas as pl
from jax.experimental.pallas import tpu as pltpu
from jax.experimental.pallas import tpu_sc as plsc


@functools.lru_cache(maxsize=None)
def _build(B, D, NC, NS):
    NW = NC * NS
    assert B % NW == 0
    b_per_w = B // NW

    mesh = plsc.VectorSubcoreMesh(core_axis_name="c", subcore_axis_name="s")

    @functools.partial(
        pl.kernel,
        mesh=mesh,
        compiler_params=pltpu.CompilerParams(use_tc_tiling_on_sc=False),
        out_type=(
            jax.ShapeDtypeStruct((B, D), jnp.float32),
            jax.ShapeDtypeStruct((B, D), jnp.float32),
        ),
        scratch_types=[
            pltpu.VMEM((b_per_w,), jnp.int32),
            pltpu.VMEM((b_per_w, D), jnp.float32),
            pltpu.VMEM((b_per_w, D), jnp.float32),
            pltpu.SemaphoreType.DMA,
            pltpu.SemaphoreType.DMA,
        ],
    )
    def k(ids_hbm, ws_hbm, wa_hbm, out_s_hbm, out_a_hbm,
          idx_v, rows_s, rows_a, sem_s, sem_a):
        wid = lax.axis_index("s") * NC + lax.axis_index("c")
        base = wid * b_per_w
        pltpu.sync_copy(ids_hbm.at[pl.ds(base, b_per_w)], idx_v)
        cp_s = pltpu.async_copy(ws_hbm.at[idx_v], rows_s, sem_s)
        cp_a = pltpu.async_copy(wa_hbm.at[idx_v], rows_a, sem_a)
        cp_s.wait()
        pltpu.sync_copy(rows_s, out_s_hbm.at[pl.ds(base, b_per_w)])
        cp_a.wait()
        pltpu.sync_copy(rows_a, out_a_hbm.at[pl.ds(base, b_per_w)])

    return k


def kernel(instance_ids, W_shape, W_appearance):
    B = instance_ids.shape[0]
    D = W_shape.shape[1]
    info = plsc.get_sparse_core_info()
    k = _build(B, D, info.num_cores, info.num_subcores)
    return k(instance_ids.astype(jnp.int32), W_shape, W_appearance)


# speedup vs baseline: 1.5344x; 1.5344x over previous
import functools

import jax
import jax.numpy as jnp
from jax import lax
from jax.experimental import pallas as pl
from jax.experimental.pallas import tpu as pltpu
from jax.experimental.pallas import tpu_sc as plsc

_K = 16


@functools.lru_cache(maxsize=None)
def _build(B, D, NC, NS):
    NW = NC * NS
    assert B % (NW * _K) == 0
    b_per_w = B // NW
    n_blk = b_per_w // _K

    mesh = plsc.VectorSubcoreMesh(core_axis_name="c", subcore_axis_name="s")

    @functools.partial(
        pl.kernel,
        mesh=mesh,
        compiler_params=pltpu.CompilerParams(needs_layout_passes=False),
        out_type=(
            jax.ShapeDtypeStruct((B, D), jnp.float32),
            jax.ShapeDtypeStruct((B, D), jnp.float32),
        ),
        scratch_types=[
            pltpu.VMEM((b_per_w,), jnp.int32),
            pltpu.VMEM((_K, D), jnp.float32),
            pltpu.VMEM((_K, D), jnp.float32),
            pltpu.SemaphoreType.DMA,
            pltpu.SemaphoreType.DMA,
        ],
    )
    def k(ids_hbm, ws_hbm, wa_hbm, out_s_hbm, out_a_hbm,
          idx_v, st_s, st_a, sem_s, sem_a):
        wid = lax.axis_index("s") * NC + lax.axis_index("c")
        base = wid * b_per_w
        pltpu.sync_copy(ids_hbm.at[pl.ds(base, b_per_w)], idx_v)

        def block(b, _):
            ids16 = idx_v[pl.ds(b * _K, _K)]
            cps = []
            for j in range(_K):
                r = ids16[j]
                cps.append(pltpu.async_copy(
                    ws_hbm.at[pl.ds(r, 1)], st_s.at[pl.ds(j, 1)], sem_s))
                cps.append(pltpu.async_copy(
                    wa_hbm.at[pl.ds(r, 1)], st_a.at[pl.ds(j, 1)], sem_a))
            for cp in cps:
                cp.wait()
            ob = base + b * _K
            pltpu.sync_copy(st_s, out_s_hbm.at[pl.ds(ob, _K)])
            pltpu.sync_copy(st_a, out_a_hbm.at[pl.ds(ob, _K)])
            return 0

        lax.fori_loop(0, n_blk, block, 0)

    return k


def kernel(instance_ids, W_shape, W_appearance):
    B = instance_ids.shape[0]
    D = W_shape.shape[1]
    info = plsc.get_sparse_core_info()
    k = _build(B, D, info.num_cores, info.num_subcores)
    return k(instance_ids.astype(jnp.int32), W_shape, W_appearance)
